# baseline (speedup 1.0000x reference)
import jax
import jax.numpy as jnp
from jax.experimental import pallas as pl

QUERY_NUM = 1000000
FEATURE_DIM = 32
BLOCK_ROWS = 8000


def _copy_body(w_ref, out_ref):
    out_ref[...] = w_ref[...][None]


def kernel(image, step, W):
    out = pl.pallas_call(
        _copy_body,
        grid=(QUERY_NUM // BLOCK_ROWS,),
        in_specs=[pl.BlockSpec((BLOCK_ROWS, FEATURE_DIM), lambda i: (i, 0))],
        out_specs=pl.BlockSpec((1, BLOCK_ROWS, FEATURE_DIM), lambda i: (0, i, 0)),
        out_shape=jax.ShapeDtypeStruct((1, QUERY_NUM, FEATURE_DIM), W.dtype),
    )(W)
    return out
